# Initial kernel scaffold; baseline (speedup 1.0000x reference)
import jax
import jax.numpy as jnp
from jax.experimental import pallas as pl


def kernel(x, edge_index, W1, b1, W2, b2):
    raise NotImplementedError("write your pallas kernel here")


# speedup vs baseline: 3.5579x; 3.5579x over previous
import functools

import jax
import jax.numpy as jnp
from jax import lax
from jax.experimental import pallas as pl
from jax.experimental.pallas import tpu as pltpu
from jax.experimental.pallas import tpu_sc as plsc

N_NODES = 10000
N_EDGES = 160000
D_IN = 256
W_HID = 512
D_OUT = 256

NC = 2
NS = 16
DH = D_IN // 2

K = 128
EPT_RAW = N_EDGES // NS
NCHUNK = -(-EPT_RAW // K)
EPT = NCHUNK * K
E_PAD = EPT * NS
NPT = 624
NPT_LAST = N_NODES - (NS - 1) * NPT
ACC_ROWS = N_NODES + 16


def _sc_aggregate(xa, xb, src, dst):
    mesh = plsc.VectorSubcoreMesh(
        core_axis_name="c", subcore_axis_name="s", num_cores=NC, num_subcores=NS
    )

    @functools.partial(
        pl.kernel,
        out_type=(
            jax.ShapeDtypeStruct((N_NODES, DH), jnp.float32),
            jax.ShapeDtypeStruct((N_NODES, DH), jnp.float32),
        ),
        mesh=mesh,
        scratch_types=[
            pltpu.VMEM((K,), jnp.int32),
            pltpu.VMEM((K,), jnp.int32),
            pltpu.VMEM((K, DH), jnp.float32),
            pltpu.VMEM_SHARED((ACC_ROWS, DH), jnp.float32),
            pltpu.SemaphoreType.DMA,
        ],
    )
    def body(xa_hbm, xb_hbm, src_hbm, dst_hbm, outa_hbm, outb_hbm,
             sidx, didx, rows, acc, sem):
        c = lax.axis_index("c")
        s = lax.axis_index("s")

        def seed(x_hbm):
            @pl.when(s < NS - 1)
            def _():
                pltpu.sync_copy(x_hbm.at[pl.ds(s * NPT, NPT)],
                                acc.at[pl.ds(s * NPT, NPT)])

            @pl.when(s == NS - 1)
            def _():
                pltpu.sync_copy(x_hbm.at[pl.ds((NS - 1) * NPT, NPT_LAST)],
                                acc.at[pl.ds((NS - 1) * NPT, NPT_LAST)])

        @pl.when(c == 0)
        def _():
            seed(xa_hbm)

        @pl.when(c == 1)
        def _():
            seed(xb_hbm)

        plsc.subcore_barrier()

        base = s * EPT

        def chunk(j, carry):
            off = pl.multiple_of(base + j * K, K)
            pltpu.sync_copy(src_hbm.at[pl.ds(off, K)], sidx)
            pltpu.sync_copy(dst_hbm.at[pl.ds(off, K)], didx)

            @pl.when(c == 0)
            def _():
                pltpu.async_copy(xa_hbm.at[sidx], rows, sem).wait()

            @pl.when(c == 1)
            def _():
                pltpu.async_copy(xb_hbm.at[sidx], rows, sem).wait()

            pltpu.sync_copy(rows, acc.at[didx], add=True)
            return carry

        lax.fori_loop(0, NCHUNK, chunk, 0)
        plsc.subcore_barrier()

        def writeback(out_hbm):
            @pl.when(s < NS - 1)
            def _():
                pltpu.sync_copy(acc.at[pl.ds(s * NPT, NPT)],
                                out_hbm.at[pl.ds(s * NPT, NPT)])

            @pl.when(s == NS - 1)
            def _():
                pltpu.sync_copy(acc.at[pl.ds((NS - 1) * NPT, NPT_LAST)],
                                out_hbm.at[pl.ds((NS - 1) * NPT, NPT_LAST)])

        @pl.when(c == 0)
        def _():
            writeback(outa_hbm)

        @pl.when(c == 1)
        def _():
            writeback(outb_hbm)

    return body(xa, xb, src, dst)


def _mlp_body(ha_ref, hb_ref, w1_ref, b1_ref, w2_ref, b2_ref, o_ref):
    h = jnp.concatenate([ha_ref[...], hb_ref[...]], axis=1)
    z = jnp.dot(h, w1_ref[...], preferred_element_type=jnp.float32) + b1_ref[...]
    z = jnp.maximum(z, 0.0)
    o_ref[...] = (
        jnp.dot(z, w2_ref[...], preferred_element_type=jnp.float32) + b2_ref[...]
    )


def _mlp(ha, hb, W1, b1, W2, b2):
    BN = 1000
    grid = (N_NODES // BN,)
    return pl.pallas_call(
        _mlp_body,
        grid=grid,
        in_specs=[
            pl.BlockSpec((BN, DH), lambda i: (i, 0)),
            pl.BlockSpec((BN, DH), lambda i: (i, 0)),
            pl.BlockSpec((D_IN, W_HID), lambda i: (0, 0)),
            pl.BlockSpec((1, W_HID), lambda i: (0, 0)),
            pl.BlockSpec((W_HID, D_OUT), lambda i: (0, 0)),
            pl.BlockSpec((1, D_OUT), lambda i: (0, 0)),
        ],
        out_specs=pl.BlockSpec((BN, D_OUT), lambda i: (i, 0)),
        out_shape=jax.ShapeDtypeStruct((N_NODES, D_OUT), jnp.float32),
    )(ha, hb, W1, b1.reshape(1, W_HID), W2, b2.reshape(1, D_OUT))


def kernel(x, edge_index, W1, b1, W2, b2):
    src = edge_index[0].astype(jnp.int32)
    dst = edge_index[1].astype(jnp.int32)
    pad = E_PAD - N_EDGES
    src = jnp.concatenate([src, jnp.zeros((pad,), jnp.int32)])
    dst = jnp.concatenate([dst, jnp.full((pad,), N_NODES, jnp.int32)])
    xa = x[:, :DH]
    xb = x[:, DH:]
    ha, hb = _sc_aggregate(xa, xb, src, dst)
    return _mlp(ha, hb, W1, b1, W2, b2)
